# Initial kernel scaffold; baseline (speedup 1.0000x reference)
import jax
import jax.numpy as jnp
from jax.experimental import pallas as pl


def kernel(x, edge_index, W_fp, b_fp, W1, b1, W2, b2, W3, b3, W_tp, b_tp, W_ih0, W_hh0, b_ih0, b_hh0, W_ih1, W_hh1, b_ih1, b_hh1, W_f1, b_f1, W_f2, b_f2):
    raise NotImplementedError("write your pallas kernel here")


# speedup vs baseline: 3.1456x; 3.1456x over previous
import functools

import jax
import jax.numpy as jnp
from jax.experimental import pallas as pl

N = 50000
F_IN = 128
SEQ_LEN = 12
H = 32
BN = 2000


def _sigmoid(z):
    return jax.nn.sigmoid(z)


def _lstm_head_kernel(xt_ref, v0_ref, k0_ref, wh0_ref, w1_ref, k1_ref,
                      wf1_ref, bf1_ref, wf2_ref, bf2_ref, y_ref):
    xt = xt_ref[...]
    v0 = v0_ref[...]
    k0 = k0_ref[...]
    wh0 = wh0_ref[...]
    w1 = w1_ref[...]
    k1 = k1_ref[...]

    zeros = jnp.zeros((xt.shape[0], H), dtype=jnp.float32)
    h0, c0, h1, c1 = zeros, zeros, zeros, zeros

    for j in range(SEQ_LEN):
        xj = jax.lax.slice(xt, (0, j), (xt.shape[0], j + 1))
        g = xj * v0 + k0 + jnp.dot(h0, wh0, preferred_element_type=jnp.float32)
        ig = _sigmoid(g[:, 0:H])
        fg = _sigmoid(g[:, H:2 * H])
        gg = jnp.tanh(g[:, 2 * H:3 * H])
        og = _sigmoid(g[:, 3 * H:4 * H])
        c0 = fg * c0 + ig * gg
        h0 = og * jnp.tanh(c0)

        hcat = jnp.concatenate([h0, h1], axis=1)
        g1 = jnp.dot(hcat, w1, preferred_element_type=jnp.float32) + k1
        i1 = _sigmoid(g1[:, 0:H])
        f1 = _sigmoid(g1[:, H:2 * H])
        gg1 = jnp.tanh(g1[:, 2 * H:3 * H])
        o1 = _sigmoid(g1[:, 3 * H:4 * H])
        c1 = f1 * c1 + i1 * gg1
        h1 = o1 * jnp.tanh(c1)

    z = jax.nn.relu(
        jnp.dot(h1, wf1_ref[...], preferred_element_type=jnp.float32)
        + bf1_ref[...])
    y = jnp.sum(z * wf2_ref[...], axis=1, keepdims=True) + bf2_ref[...]
    y_ref[...] = y


def kernel(x, edge_index, W_fp, b_fp, W1, b1, W2, b2, W3, b3, W_tp, b_tp,
           W_ih0, W_hh0, b_ih0, b_hh0, W_ih1, W_hh1, b_ih1, b_hh1,
           W_f1, b_f1, W_f2, b_f2):
    x_tail = jax.lax.slice(x, (0, F_IN - SEQ_LEN), (N, F_IN))

    v0 = W_tp @ W_ih0.T
    k0 = (b_tp @ W_ih0.T + b_ih0 + b_hh0)[None, :]
    wh0 = W_hh0.T
    w1 = jnp.concatenate([W_ih1, W_hh1], axis=1).T
    k1 = (b_ih1 + b_hh1)[None, :]
    bf1 = b_f1[None, :]
    wf2 = W_f2.T
    bf2 = b_f2[None, :]

    grid = (N // BN,)
    full = lambda i: (0, 0)
    y = pl.pallas_call(
        _lstm_head_kernel,
        grid=grid,
        in_specs=[
            pl.BlockSpec((BN, SEQ_LEN), lambda i: (i, 0)),
            pl.BlockSpec((1, 4 * H), full),
            pl.BlockSpec((1, 4 * H), full),
            pl.BlockSpec((H, 4 * H), full),
            pl.BlockSpec((2 * H, 4 * H), full),
            pl.BlockSpec((1, 4 * H), full),
            pl.BlockSpec((H, 16), full),
            pl.BlockSpec((1, 16), full),
            pl.BlockSpec((1, 16), full),
            pl.BlockSpec((1, 1), full),
        ],
        out_specs=pl.BlockSpec((BN, 1), lambda i: (i, 0)),
        out_shape=jax.ShapeDtypeStruct((N, 1), jnp.float32),
    )(x_tail, v0, k0, wh0, w1, k1, W_f1, bf1, wf2, bf2)
    return y


# speedup vs baseline: 3.3578x; 1.0675x over previous
import jax
import jax.numpy as jnp
from jax.experimental import pallas as pl

N = 50000
F_IN = 128
SEQ_LEN = 12
H = 32
BN = 2000


def _lstm_head_kernel(xt_ref, wbd_ref, k0_ref, wh0_ref, w1_ref, k1_ref,
                      amul_ref, aadd_ref, wf1_ref, bf1_ref, wf2_ref, bf2_ref,
                      y_ref):
    xt = xt_ref[...]
    k0 = k0_ref[...]
    wh0 = wh0_ref[...]
    w1 = w1_ref[...]
    k1 = k1_ref[...]
    amul = amul_ref[...]
    aadd = aadd_ref[...]

    g0_all = jnp.dot(xt, wbd_ref[...], preferred_element_type=jnp.float32)

    zeros = jnp.zeros((xt.shape[0], H), dtype=jnp.float32)
    h0, c0, h1, c1 = zeros, zeros, zeros, zeros

    for j in range(SEQ_LEN):
        g = (g0_all[:, j * 4 * H:(j + 1) * 4 * H] + k0
             + jnp.dot(h0, wh0, preferred_element_type=jnp.float32))
        a = jax.nn.sigmoid(g) * amul + aadd
        c0 = a[:, H:2 * H] * c0 + a[:, 0:H] * a[:, 2 * H:3 * H]
        h0 = a[:, 3 * H:4 * H] * jnp.tanh(c0)

        hcat = jnp.concatenate([h0, h1], axis=1)
        g1 = jnp.dot(hcat, w1, preferred_element_type=jnp.float32) + k1
        a1 = jax.nn.sigmoid(g1) * amul + aadd
        c1 = a1[:, H:2 * H] * c1 + a1[:, 0:H] * a1[:, 2 * H:3 * H]
        h1 = a1[:, 3 * H:4 * H] * jnp.tanh(c1)

    z = jax.nn.relu(
        jnp.dot(h1, wf1_ref[...], preferred_element_type=jnp.float32)
        + bf1_ref[...])
    y = jnp.sum(z * wf2_ref[...], axis=1, keepdims=True) + bf2_ref[...]
    y_ref[...] = y


def kernel(x, edge_index, W_fp, b_fp, W1, b1, W2, b2, W3, b3, W_tp, b_tp,
           W_ih0, W_hh0, b_ih0, b_hh0, W_ih1, W_hh1, b_ih1, b_hh1,
           W_f1, b_f1, W_f2, b_f2):
    x_tail = jax.lax.slice(x, (0, F_IN - SEQ_LEN), (N, F_IN))

    ones = jnp.ones((H,), jnp.float32)
    sc = jnp.concatenate([ones, ones, 2.0 * ones, ones])[None, :]
    amul = jnp.concatenate([ones, ones, 2.0 * ones, ones])[None, :]
    aadd = jnp.concatenate([0 * ones, 0 * ones, -ones, 0 * ones])[None, :]

    v0 = (W_tp @ W_ih0.T) * sc
    wbd = jnp.kron(jnp.eye(SEQ_LEN, dtype=jnp.float32), v0)
    k0 = ((b_tp @ W_ih0.T + b_ih0 + b_hh0)[None, :]) * sc
    wh0 = W_hh0.T * sc
    w1 = jnp.concatenate([W_ih1, W_hh1], axis=1).T * sc
    k1 = ((b_ih1 + b_hh1)[None, :]) * sc
    bf1 = b_f1[None, :]
    wf2 = W_f2.T
    bf2 = b_f2[None, :]

    grid = (N // BN,)
    full = lambda i: (0, 0)
    y = pl.pallas_call(
        _lstm_head_kernel,
        grid=grid,
        in_specs=[
            pl.BlockSpec((BN, SEQ_LEN), lambda i: (i, 0)),
            pl.BlockSpec((SEQ_LEN, SEQ_LEN * 4 * H), full),
            pl.BlockSpec((1, 4 * H), full),
            pl.BlockSpec((H, 4 * H), full),
            pl.BlockSpec((2 * H, 4 * H), full),
            pl.BlockSpec((1, 4 * H), full),
            pl.BlockSpec((1, 4 * H), full),
            pl.BlockSpec((1, 4 * H), full),
            pl.BlockSpec((H, 16), full),
            pl.BlockSpec((1, 16), full),
            pl.BlockSpec((1, 16), full),
            pl.BlockSpec((1, 1), full),
        ],
        out_specs=pl.BlockSpec((BN, 1), lambda i: (i, 0)),
        out_shape=jax.ShapeDtypeStruct((N, 1), jnp.float32),
    )(x_tail, wbd, k0, wh0, w1, k1, amul, aadd, W_f1, bf1, wf2, bf2)
    return y


# speedup vs baseline: 7.7470x; 2.3072x over previous
import jax
import jax.numpy as jnp
from jax.experimental import pallas as pl

N = 50000
NPAD = 51200
F_IN = 128
SEQ_LEN = 12
H = 32
G = 4
BN = 2048
BP = BN // G
NBLK = NPAD // BN
HG = H * G


def _lstm_head_kernel(xt_ref, wbd_ref, wh0_ref, w1_ref, k1_ref,
                      wf1_ref, bf1_ref, wf2_ref, bf2_ref, y_ref):
    xt = xt_ref[...]
    wh0 = wh0_ref[...]
    w1 = w1_ref[...]
    k1 = k1_ref[...]

    gin = jnp.dot(xt, wbd_ref[...], preferred_element_type=jnp.float32)

    zeros = jnp.zeros((xt.shape[0], HG), dtype=jnp.float32)
    h0, c0, h1, c1 = zeros, zeros, zeros, zeros
    W4 = 4 * HG

    for j in range(SEQ_LEN):
        g = (gin[:, j * W4:(j + 1) * W4]
             + jnp.dot(h0, wh0, preferred_element_type=jnp.float32))
        a = jax.nn.sigmoid(g)
        gt = 2.0 * a[:, 2 * HG:3 * HG] - 1.0
        c0 = a[:, HG:2 * HG] * c0 + a[:, 0:HG] * gt
        h0 = a[:, 3 * HG:4 * HG] * jnp.tanh(c0)

        hcat = jnp.concatenate([h0, h1], axis=1)
        g1 = jnp.dot(hcat, w1, preferred_element_type=jnp.float32) + k1
        a1 = jax.nn.sigmoid(g1)
        gt1 = 2.0 * a1[:, 2 * HG:3 * HG] - 1.0
        c1 = a1[:, HG:2 * HG] * c1 + a1[:, 0:HG] * gt1
        h1 = a1[:, 3 * HG:4 * HG] * jnp.tanh(c1)

    z = jax.nn.relu(
        jnp.dot(h1, wf1_ref[...], preferred_element_type=jnp.float32)
        + bf1_ref[...])
    y = jnp.dot(z, wf2_ref[...], preferred_element_type=jnp.float32)
    y_ref[...] = y + bf2_ref[...]


def kernel(x, edge_index, W_fp, b_fp, W1, b1, W2, b2, W3, b3, W_tp, b_tp,
           W_ih0, W_hh0, b_ih0, b_hh0, W_ih1, W_hh1, b_ih1, b_hh1,
           W_f1, b_f1, W_f2, b_f2):
    f32 = jnp.float32
    x_tail = jax.lax.slice(x, (0, F_IN - SEQ_LEN), (N, F_IN))
    x_tail = jnp.pad(x_tail, ((0, NPAD - N), (0, 0)))

    xp = x_tail.reshape(NBLK, G, BP, SEQ_LEN).transpose(0, 2, 1, 3)
    xp = xp.reshape(NPAD // G, SEQ_LEN * G)
    xp = jnp.concatenate([xp, jnp.ones((NPAD // G, 1), f32)], axis=1)

    I4 = jnp.eye(G, dtype=f32)
    I12 = jnp.eye(SEQ_LEN, dtype=f32)
    ones32 = jnp.ones((H,), f32)
    sc = jnp.concatenate([ones32, ones32, 2.0 * ones32, ones32])[None, :]

    v0 = ((W_tp @ W_ih0.T) * sc).reshape(4, H)
    k0 = (((b_tp @ W_ih0.T + b_ih0 + b_hh0)[None, :]) * sc).reshape(4, H)
    wbd = jnp.einsum('jk,gh,bu->gjkbhu', I12, I4, v0).reshape(
        SEQ_LEN * G, SEQ_LEN * 4 * HG)
    k0row = jnp.broadcast_to(k0[None, :, None, :],
                             (SEQ_LEN, 4, G, H)).reshape(1, SEQ_LEN * 4 * HG)
    wbd = jnp.concatenate([wbd, k0row], axis=0)

    wh0s = (W_hh0.T * sc).reshape(H, 4, H)
    wh0b = jnp.einsum('gh,kbu->gkbhu', I4, wh0s).reshape(HG, 4 * HG)

    w1s = (jnp.concatenate([W_ih1, W_hh1], axis=1).T * sc)
    w1a = w1s[0:H].reshape(H, 4, H)
    w1b = w1s[H:2 * H].reshape(H, 4, H)
    w1big = jnp.concatenate([
        jnp.einsum('gh,kbu->gkbhu', I4, w1a).reshape(HG, 4 * HG),
        jnp.einsum('gh,kbu->gkbhu', I4, w1b).reshape(HG, 4 * HG),
    ], axis=0)
    k1r = ((b_ih1 + b_hh1) * sc[0]).reshape(4, H)
    k1big = jnp.broadcast_to(k1r[None, :, None, :],
                             (1, 4, G, H)).reshape(1, 4 * HG)

    wf1b = jnp.einsum('gh,ku->gkhu', I4, W_f1).reshape(HG, 16 * G)
    bf1b = jnp.broadcast_to(b_f1[None, None, :], (1, G, 16)).reshape(1, 16 * G)
    wf2b = jnp.einsum('gh,u->guh', I4, W_f2[:, 0]).reshape(16 * G, G)
    bf2b = b_f2[None, :]

    full = lambda i: (0, 0)
    yp = pl.pallas_call(
        _lstm_head_kernel,
        grid=(NBLK,),
        in_specs=[
            pl.BlockSpec((BP, SEQ_LEN * G + 1), lambda i: (i, 0)),
            pl.BlockSpec(wbd.shape, full),
            pl.BlockSpec(wh0b.shape, full),
            pl.BlockSpec(w1big.shape, full),
            pl.BlockSpec(k1big.shape, full),
            pl.BlockSpec(wf1b.shape, full),
            pl.BlockSpec(bf1b.shape, full),
            pl.BlockSpec(wf2b.shape, full),
            pl.BlockSpec(bf2b.shape, full),
        ],
        out_specs=pl.BlockSpec((BP, G), lambda i: (i, 0)),
        out_shape=jax.ShapeDtypeStruct((NPAD // G, G), f32),
    )(xp, wbd, wh0b, w1big, k1big, wf1b, bf1b, wf2b, bf2b)

    y = yp.reshape(NBLK, BP, G).transpose(0, 2, 1).reshape(NPAD, 1)
    return jax.lax.slice(y, (0, 0), (N, 1))


# speedup vs baseline: 10.1119x; 1.3053x over previous
import jax
import jax.numpy as jnp
from jax.experimental import pallas as pl

N = 50000
NPAD = 51200
F_IN = 128
SEQ_LEN = 12
H = 32
G = 4
BN = 2048
BP = BN // G
NBLK = NPAD // BN
HG = H * G
W4 = 4 * HG


def _lstm_head_kernel(xt_ref, wbd_ref, k0_ref, wh0_ref, w1_ref, k1_ref,
                      wf1_ref, bf1_ref, wf2_ref, bf2_ref, y_ref):
    xt = xt_ref[...]
    k0 = k0_ref[...]
    wh0 = wh0_ref[...]
    w1 = w1_ref[...]
    k1 = k1_ref[...]

    gin = jnp.dot(xt, wbd_ref[...], preferred_element_type=jnp.float32)

    zeros = jnp.zeros((xt.shape[0], HG), dtype=jnp.float32)
    h0, c0, h1, c1 = zeros, zeros, zeros, zeros

    for j in range(SEQ_LEN):
        g = (gin[:, j * W4:(j + 1) * W4] + k0
             + jnp.dot(h0, wh0, preferred_element_type=jnp.float32))
        a = jnp.tanh(g)
        si = 0.5 * a[:, 0:HG] + 0.5
        sf = 0.5 * a[:, HG:2 * HG] + 0.5
        so = 0.5 * a[:, 3 * HG:4 * HG] + 0.5
        c0 = sf * c0 + si * a[:, 2 * HG:3 * HG]
        h0 = so * jnp.tanh(c0)

        hcat = jnp.concatenate([h0, h1], axis=1)
        g1 = jnp.dot(hcat, w1, preferred_element_type=jnp.float32) + k1
        a1 = jnp.tanh(g1)
        si1 = 0.5 * a1[:, 0:HG] + 0.5
        sf1 = 0.5 * a1[:, HG:2 * HG] + 0.5
        so1 = 0.5 * a1[:, 3 * HG:4 * HG] + 0.5
        c1 = sf1 * c1 + si1 * a1[:, 2 * HG:3 * HG]
        h1 = so1 * jnp.tanh(c1)

    z = jax.nn.relu(
        jnp.dot(h1, wf1_ref[...], preferred_element_type=jnp.float32)
        + bf1_ref[...])
    y = jnp.dot(z, wf2_ref[...], preferred_element_type=jnp.float32)
    y_ref[...] = y + bf2_ref[...]


def kernel(x, edge_index, W_fp, b_fp, W1, b1, W2, b2, W3, b3, W_tp, b_tp,
           W_ih0, W_hh0, b_ih0, b_hh0, W_ih1, W_hh1, b_ih1, b_hh1,
           W_f1, b_f1, W_f2, b_f2):
    f32 = jnp.float32
    x_tail = jax.lax.slice(x, (0, F_IN - SEQ_LEN), (N, F_IN))
    x_tail = jnp.pad(x_tail, ((0, NPAD - N), (0, 0)))

    xp = x_tail.reshape(NPAD // G, SEQ_LEN * G)

    I4 = jnp.eye(G, dtype=f32)
    I12 = jnp.eye(SEQ_LEN, dtype=f32)
    half32 = jnp.full((H,), 0.5, f32)
    one32 = jnp.ones((H,), f32)
    sc = jnp.concatenate([half32, half32, one32, half32])[None, :]

    v0 = ((W_tp @ W_ih0.T) * sc).reshape(4, H)
    k0 = (((b_tp @ W_ih0.T + b_ih0 + b_hh0)[None, :]) * sc)
    k0big = jnp.broadcast_to(k0.reshape(4, 1, H),
                             (4, G, H)).reshape(1, W4)
    wbd = jnp.einsum('jk,gh,bu->gjkbhu', I12, I4, v0).reshape(
        SEQ_LEN * G, SEQ_LEN * W4)

    wh0s = (W_hh0.T * sc).reshape(H, 4, H)
    wh0b = jnp.einsum('gh,kbu->gkbhu', I4, wh0s).reshape(HG, W4)

    w1s = (jnp.concatenate([W_ih1, W_hh1], axis=1).T * sc)
    w1a = w1s[0:H].reshape(H, 4, H)
    w1b = w1s[H:2 * H].reshape(H, 4, H)
    w1big = jnp.concatenate([
        jnp.einsum('gh,kbu->gkbhu', I4, w1a).reshape(HG, W4),
        jnp.einsum('gh,kbu->gkbhu', I4, w1b).reshape(HG, W4),
    ], axis=0)
    k1 = (((b_ih1 + b_hh1)[None, :]) * sc)
    k1big = jnp.broadcast_to(k1.reshape(4, 1, H),
                             (4, G, H)).reshape(1, W4)

    wf1b = jnp.einsum('gh,ku->gkhu', I4, W_f1).reshape(HG, 16 * G)
    bf1b = jnp.broadcast_to(b_f1[None, None, :], (1, G, 16)).reshape(1, 16 * G)
    wf2b = jnp.einsum('gh,u->guh', I4, W_f2[:, 0]).reshape(16 * G, G)
    bf2b = b_f2[None, :]

    full = lambda i: (0, 0)
    yp = pl.pallas_call(
        _lstm_head_kernel,
        grid=(NBLK,),
        in_specs=[
            pl.BlockSpec((BP, SEQ_LEN * G), lambda i: (i, 0)),
            pl.BlockSpec(wbd.shape, full),
            pl.BlockSpec(k0big.shape, full),
            pl.BlockSpec(wh0b.shape, full),
            pl.BlockSpec(w1big.shape, full),
            pl.BlockSpec(k1big.shape, full),
            pl.BlockSpec(wf1b.shape, full),
            pl.BlockSpec(bf1b.shape, full),
            pl.BlockSpec(wf2b.shape, full),
            pl.BlockSpec(bf2b.shape, full),
        ],
        out_specs=pl.BlockSpec((BP, G), lambda i: (i, 0)),
        out_shape=jax.ShapeDtypeStruct((NPAD // G, G), f32),
    )(xp, wbd, k0big, wh0b, w1big, k1big, wf1b, bf1b, wf2b, bf2b)

    y = yp.reshape(NPAD, 1)
    return jax.lax.slice(y, (0, 0), (N, 1))
